# Initial kernel scaffold; baseline (speedup 1.0000x reference)
import jax
import jax.numpy as jnp
from jax.experimental import pallas as pl


def kernel(seqs, table):
    raise NotImplementedError("write your pallas kernel here")


# speedup vs baseline: 2.2434x; 2.2434x over previous
import functools

import jax
import jax.numpy as jnp
from jax import lax
from jax.experimental import pallas as pl
from jax.experimental.pallas import tpu as pltpu
from jax.experimental.pallas import tpu_sc as plsc

_NC = 2
_NS = 16
_NW = _NC * _NS
_CH = 128


@functools.cache
def _build(V, D, n_chunks):
    per_w = n_chunks * _CH
    mesh = plsc.VectorSubcoreMesh(core_axis_name="c", subcore_axis_name="s")

    @functools.partial(
        pl.kernel,
        out_type=jax.ShapeDtypeStruct((_NW * per_w, D), jnp.float32),
        mesh=mesh,
        scratch_types=[
            pltpu.VMEM((n_chunks + 1, _CH), jnp.int32),
            pltpu.VMEM((_CH, D), jnp.float32),
            pltpu.VMEM((_CH, D), jnp.float32),
            pltpu.SemaphoreType.DMA,
            pltpu.SemaphoreType.DMA,
        ],
    )
    def k(idx_hbm, table_hbm, out_hbm, idx_v, rows0, rows1, sem0, sem1):
        wid = lax.axis_index("s") * _NC + lax.axis_index("c")
        base = wid * per_w

        def wait_gather(rows, sem):
            pltpu.make_async_copy(table_hbm.at[pl.ds(0, _CH)], rows, sem).wait()

        pltpu.sync_copy(idx_hbm.at[wid], idx_v)
        pltpu.async_copy(table_hbm.at[idx_v.at[0]], rows0, sem0)

        @pl.loop(0, n_chunks, step=2)
        def _(i):
            pltpu.async_copy(table_hbm.at[idx_v.at[i + 1]], rows1, sem1)
            wait_gather(rows0, sem0)
            pltpu.sync_copy(rows0, out_hbm.at[pl.ds(base + i * _CH, _CH)])
            pltpu.async_copy(table_hbm.at[idx_v.at[i + 2]], rows0, sem0)
            wait_gather(rows1, sem1)
            pltpu.sync_copy(rows1, out_hbm.at[pl.ds(base + (i + 1) * _CH, _CH)])

        wait_gather(rows0, sem0)

    return k


def kernel(seqs, table):
    B, H = seqs.shape
    V, D = table.shape
    flat = seqs.reshape(-1).astype(jnp.int32)
    n = flat.shape[0]
    assert n % (_NW * _CH) == 0 and n // (_NW * _CH) % 2 == 0
    n_chunks = n // (_NW * _CH)
    idx = flat.reshape(_NW, n_chunks, _CH)
    idx = jnp.pad(idx, ((0, 0), (0, 1), (0, 0)))
    out = _build(V, D, n_chunks)(idx, table)
    return out.reshape(B, H, D)
